# baseline (speedup 1.0000x reference)
import functools

import jax
import jax.numpy as jnp
from jax.experimental import pallas as pl
from jax.experimental.pallas import tpu as pltpu

_IN = 1024
_OUT = 4096
_K = max(1, int(_OUT * 0.05))
_ROWS = 8192
_BLK = 256


def _body(x_ref, wrt_ref, brt_ref, wt_ref, b_ref, out_ref, f_ref):
    xb = x_ref[...]
    le = (
        jnp.dot(xb, wrt_ref[...], preferred_element_type=jnp.float32)
        + brt_ref[...]
    )
    logits = le[:, :_OUT]
    s1 = le[:, _OUT:_OUT + 1]
    f_ref[...] = logits

    kf = jnp.float32(_K)
    rmax = jnp.max(logits, axis=1, keepdims=True)
    rmin = jnp.min(logits, axis=1, keepdims=True)
    n = jnp.float32(_OUT)
    mean = s1 / n
    s2 = jnp.sum(logits * logits, axis=1, keepdims=True)
    sd = jnp.sqrt(jnp.maximum(s2 / n - mean * mean, 0.0))
    hi0 = rmax + (jnp.abs(rmax) * jnp.float32(2.0**-22) + jnp.float32(1e-35))

    def update(state, mid, cnt):
        lo, hi, cl, ch = state
        eq = cnt == kf
        ge = cnt >= kf
        lo = jnp.where(ge, mid, lo)
        cl = jnp.where(ge, cnt, cl)
        hi = jnp.where(eq, mid, jnp.where(ge, hi, mid))
        ch = jnp.where(ge, ch, cnt)
        return lo, hi, cl, ch

    def count(t):
        return jnp.sum(
            (f_ref[...] >= t).astype(jnp.float32), axis=1, keepdims=True
        )

    t0 = mean + jnp.float32(1.6448536) * sd
    state = update((rmin, hi0, n, jnp.zeros_like(rmin)), t0, count(t0))
    lo, hi, cl, ch = state

    def step(state, bisect):
        lo, hi, cl, ch = state
        if bisect:
            mid = 0.5 * lo + 0.5 * hi
        else:
            frac = (cl - kf) / jnp.maximum(cl - ch, 1.0)
            frac = jnp.clip(frac, 0.03, 0.97)
            mid = lo + (hi - lo) * frac
        return update(state, mid, count(mid))

    def chunk(state):
        for _u in range(3):
            state = step(state, False)
        return step(state, True)

    def cond(carry):
        i, state = carry
        return jnp.logical_and(i < 10, jnp.any(state[0] < state[1]))

    def body(carry):
        i, state = carry
        return i + 1, chunk(state)

    state = chunk((lo, hi, cl, ch))
    _, (lo, _, _, _) = jax.lax.while_loop(cond, body, (jnp.int32(0), state))

    mask = f_ref[...] >= lo
    out = (
        jnp.dot(xb, wt_ref[...], preferred_element_type=jnp.float32)
        + b_ref[...]
    )
    out_ref[...] = jnp.where(mask, out, 0.0)


@functools.partial(jax.jit, static_argnames=())
def kernel(x, W_router, b_router, W, b):
    bsz, seq, din = x.shape
    xf = x.reshape(_ROWS, din)
    wrt = W_router.T
    wrt = jnp.concatenate(
        [wrt, jnp.sum(wrt, axis=1, keepdims=True),
         jnp.zeros((din, 127), jnp.float32)], axis=1)
    wt = W.T
    brt = jnp.concatenate(
        [b_router, jnp.sum(b_router, keepdims=True),
         jnp.zeros((127,), jnp.float32)]).reshape(1, _OUT + 128)
    bb = b.reshape(1, _OUT)

    grid = (_ROWS // _BLK,)
    out = pl.pallas_call(
        _body,
        grid=grid,
        in_specs=[
            pl.BlockSpec((_BLK, din), lambda i: (i, 0)),
            pl.BlockSpec((din, _OUT + 128), lambda i: (0, 0)),
            pl.BlockSpec((1, _OUT + 128), lambda i: (0, 0)),
            pl.BlockSpec((din, _OUT), lambda i: (0, 0)),
            pl.BlockSpec((1, _OUT), lambda i: (0, 0)),
        ],
        out_specs=pl.BlockSpec((_BLK, _OUT), lambda i: (i, 0)),
        out_shape=jax.ShapeDtypeStruct((_ROWS, _OUT), jnp.float32),
        scratch_shapes=[pltpu.VMEM((_BLK, _OUT), jnp.float32)],
    )(xf, wrt, brt, wt, bb)
    return out.reshape(bsz, seq, _OUT)


# speedup vs baseline: 1.0302x; 1.0302x over previous
import functools

import jax
import jax.numpy as jnp
from jax.experimental import pallas as pl
from jax.experimental.pallas import tpu as pltpu

_IN = 1024
_OUT = 4096
_K = max(1, int(_OUT * 0.05))
_ROWS = 8192
_BLK = 128


def _body(x_ref, wrt_ref, brt_ref, wt_ref, b_ref, out_ref, f_ref):
    xb = x_ref[...]
    logits = (
        jnp.dot(xb, wrt_ref[...], preferred_element_type=jnp.float32)
        + brt_ref[...]
    )
    f_ref[...] = logits

    kf = jnp.float32(_K)
    rmax = jnp.max(logits, axis=1, keepdims=True)
    rmin = jnp.min(logits, axis=1, keepdims=True)
    s1 = jnp.sum(logits, axis=1, keepdims=True)
    s2 = jnp.sum(logits * logits, axis=1, keepdims=True)
    n = jnp.float32(logits.shape[1])
    mean = s1 / n
    sd = jnp.sqrt(jnp.maximum(s2 / n - mean * mean, 0.0))
    hi0 = rmax + (jnp.abs(rmax) * jnp.float32(2.0**-22) + jnp.float32(1e-35))

    t0 = mean + jnp.float32(1.6448536) * sd
    cnt0 = jnp.sum((logits >= t0).astype(jnp.float32), axis=1, keepdims=True)
    eq0 = cnt0 == kf
    ge0 = cnt0 >= kf
    lo = jnp.where(ge0, t0, rmin)
    cl = jnp.where(ge0, cnt0, n)
    hi = jnp.where(eq0, t0, jnp.where(ge0, hi0, t0))
    ch = jnp.where(ge0, jnp.float32(0.0), cnt0)

    def step(state, bisect):
        lo, hi, cl, ch = state
        if bisect:
            mid = 0.5 * lo + 0.5 * hi
        else:
            frac = (cl - kf) / jnp.maximum(cl - ch, 1.0)
            frac = jnp.clip(frac, 0.03, 0.97)
            mid = lo + (hi - lo) * frac
        cnt = jnp.sum(
            (f_ref[...] >= mid).astype(jnp.float32), axis=1, keepdims=True
        )
        eq = cnt == kf
        ge = cnt >= kf
        lo = jnp.where(ge, mid, lo)
        cl = jnp.where(ge, cnt, cl)
        hi = jnp.where(eq, mid, jnp.where(ge, hi, mid))
        ch = jnp.where(ge, ch, cnt)
        return lo, hi, cl, ch

    def chunk(state):
        for _u in range(3):
            state = step(state, False)
        return step(state, True)

    def cond(carry):
        i, state = carry
        return jnp.logical_and(i < 10, jnp.any(state[0] < state[1]))

    def body(carry):
        i, state = carry
        return i + 1, chunk(state)

    state = chunk((lo, hi, cl, ch))
    _, (lo, _, _, _) = jax.lax.while_loop(cond, body, (jnp.int32(0), state))

    mask = f_ref[...] >= lo
    out = (
        jnp.dot(xb, wt_ref[...], preferred_element_type=jnp.float32)
        + b_ref[...]
    )
    out_ref[...] = jnp.where(mask, out, 0.0)


@functools.partial(jax.jit, static_argnames=())
def kernel(x, W_router, b_router, W, b):
    bsz, seq, din = x.shape
    xf = x.reshape(_ROWS, din)
    wrt = W_router.T
    wt = W.T
    brt = b_router.reshape(1, _OUT)
    bb = b.reshape(1, _OUT)

    grid = (_ROWS // _BLK,)
    out = pl.pallas_call(
        _body,
        grid=grid,
        in_specs=[
            pl.BlockSpec((_BLK, din), lambda i: (i, 0)),
            pl.BlockSpec((din, _OUT), lambda i: (0, 0)),
            pl.BlockSpec((1, _OUT), lambda i: (0, 0)),
            pl.BlockSpec((din, _OUT), lambda i: (0, 0)),
            pl.BlockSpec((1, _OUT), lambda i: (0, 0)),
        ],
        out_specs=pl.BlockSpec((_BLK, _OUT), lambda i: (i, 0)),
        out_shape=jax.ShapeDtypeStruct((_ROWS, _OUT), jnp.float32),
        scratch_shapes=[pltpu.VMEM((_BLK, _OUT), jnp.float32)],
    )(xf, wrt, brt, wt, bb)
    return out.reshape(bsz, seq, _OUT)


# speedup vs baseline: 1.1245x; 1.0915x over previous
import functools

import jax
import jax.numpy as jnp
from jax.experimental import pallas as pl
from jax.experimental.pallas import tpu as pltpu

_IN = 1024
_OUT = 4096
_K = max(1, int(_OUT * 0.05))
_ROWS = 8192
_BLK = 256


def _body(x_ref, wrt_ref, brt_ref, wt_ref, b_ref, out_ref, f_ref):
    xb = x_ref[...]
    logits = (
        jnp.dot(xb, wrt_ref[...], preferred_element_type=jnp.float32)
        + brt_ref[...]
    )
    f_ref[...] = logits

    kf = jnp.float32(_K)
    rmax = jnp.max(logits, axis=1, keepdims=True)
    rmin = jnp.min(logits, axis=1, keepdims=True)
    s1 = jnp.sum(logits, axis=1, keepdims=True)
    s2 = jnp.sum(logits * logits, axis=1, keepdims=True)
    n = jnp.float32(logits.shape[1])
    mean = s1 / n
    sd = jnp.sqrt(jnp.maximum(s2 / n - mean * mean, 0.0))
    hi0 = rmax + (jnp.abs(rmax) * jnp.float32(2.0**-22) + jnp.float32(1e-35))

    t0 = mean + jnp.float32(1.6448536) * sd
    cnt0 = jnp.sum((logits >= t0).astype(jnp.float32), axis=1, keepdims=True)
    eq0 = cnt0 == kf
    ge0 = cnt0 >= kf
    lo = jnp.where(ge0, t0, rmin)
    cl = jnp.where(ge0, cnt0, n)
    hi = jnp.where(eq0, t0, jnp.where(ge0, hi0, t0))
    ch = jnp.where(ge0, jnp.float32(0.0), cnt0)

    def step(state, bisect):
        lo, hi, cl, ch = state
        if bisect:
            mid = 0.5 * lo + 0.5 * hi
        else:
            frac = (cl - kf) / jnp.maximum(cl - ch, 1.0)
            frac = jnp.clip(frac, 0.03, 0.97)
            mid = lo + (hi - lo) * frac
        cnt = jnp.sum(
            (f_ref[...] >= mid).astype(jnp.float32), axis=1, keepdims=True
        )
        eq = cnt == kf
        ge = cnt >= kf
        lo = jnp.where(ge, mid, lo)
        cl = jnp.where(ge, cnt, cl)
        hi = jnp.where(eq, mid, jnp.where(ge, hi, mid))
        ch = jnp.where(ge, ch, cnt)
        return lo, hi, cl, ch

    def chunk(state, ni):
        for _u in range(ni):
            state = step(state, False)
        return step(state, True)

    def cond(carry):
        i, state = carry
        return jnp.logical_and(i < 10, jnp.any(state[0] < state[1]))

    def body(carry):
        i, state = carry
        return i + 1, chunk(state, 3)

    state = chunk((lo, hi, cl, ch), 5)
    _, (lo, _, _, _) = jax.lax.while_loop(cond, body, (jnp.int32(0), state))

    mask = f_ref[...] >= lo
    out = (
        jnp.dot(xb, wt_ref[...], preferred_element_type=jnp.float32)
        + b_ref[...]
    )
    out_ref[...] = jnp.where(mask, out, 0.0)


@functools.partial(jax.jit, static_argnames=())
def kernel(x, W_router, b_router, W, b):
    bsz, seq, din = x.shape
    xf = x.reshape(_ROWS, din)
    wrt = W_router.T
    wt = W.T
    brt = b_router.reshape(1, _OUT)
    bb = b.reshape(1, _OUT)

    grid = (_ROWS // _BLK,)
    out = pl.pallas_call(
        _body,
        grid=grid,
        in_specs=[
            pl.BlockSpec((_BLK, din), lambda i: (i, 0)),
            pl.BlockSpec((din, _OUT), lambda i: (0, 0)),
            pl.BlockSpec((1, _OUT), lambda i: (0, 0)),
            pl.BlockSpec((din, _OUT), lambda i: (0, 0)),
            pl.BlockSpec((1, _OUT), lambda i: (0, 0)),
        ],
        out_specs=pl.BlockSpec((_BLK, _OUT), lambda i: (i, 0)),
        out_shape=jax.ShapeDtypeStruct((_ROWS, _OUT), jnp.float32),
        scratch_shapes=[pltpu.VMEM((_BLK, _OUT), jnp.float32)],
    )(xf, wrt, brt, wt, bb)
    return out.reshape(bsz, seq, _OUT)
